# Initial kernel scaffold; baseline (speedup 1.0000x reference)
import jax
import jax.numpy as jnp
from jax.experimental import pallas as pl


def kernel(y_pred):
    raise NotImplementedError("write your pallas kernel here")


# speedup vs baseline: 1.1793x; 1.1793x over previous
import jax
import jax.numpy as jnp
from jax.experimental import pallas as pl
from jax.experimental.pallas import tpu as pltpu

N_CLASSES = 81
TOP_K = 200
CONF_THRESH = 0.01
IOU_THRESH = 0.45
IMG_H = 512.0
IMG_W = 512.0
ROWS = 8
CHUNK = 1000


def _decode_body(y_ref, s_ref, x1_ref, y1_ref, x2_ref, y2_ref, cl_ref):
    y = y_ref[0]
    ycls = y[:, :N_CLASSES]
    conf = jnp.max(ycls, axis=1)
    colio = jax.lax.broadcasted_iota(jnp.int32, (CHUNK, N_CLASSES), 1)
    cls = jnp.min(jnp.where(ycls == conf[:, None], colio, N_CLASSES), axis=1)
    c81 = y[:, 81]; c82 = y[:, 82]; c83 = y[:, 83]; c84 = y[:, 84]
    c85 = y[:, 85]; c86 = y[:, 86]; c87 = y[:, 87]; c88 = y[:, 88]
    c89 = y[:, 89]; c90 = y[:, 90]; c91 = y[:, 91]; c92 = y[:, 92]
    cx = c81 * c89 * c87 + c85
    cy = c82 * c90 * c88 + c86
    w = jnp.exp(c83 * c91) * c87
    h = jnp.exp(c84 * c92) * c88
    valid = (cls != 0) & (conf > CONF_THRESH)
    s_ref[0, 0, 0, :] = jnp.where(valid, conf, -1.0)
    x1_ref[0, 0, 0, :] = (cx - 0.5 * w) * IMG_W
    y1_ref[0, 0, 0, :] = (cy - 0.5 * h) * IMG_H
    x2_ref[0, 0, 0, :] = (cx + 0.5 * w) * IMG_W
    y2_ref[0, 0, 0, :] = (cy + 0.5 * h) * IMG_H
    cl_ref[0, 0, 0, :] = cls.astype(jnp.float32)


def _nms_body(s_in, x1_in, y1_in, x2_in, y2_in, cl_in,
              ocls_ref, oconf_ref, ox1_ref, oy1_ref, ox2_ref, oy2_ref,
              s_ref, ar_ref):
    cols = s_in.shape[2]
    n = ROWS * cols
    s_ref[:, :] = s_in[0]
    x1 = x1_in[0]
    y1 = y1_in[0]
    x2 = x2_in[0]
    y2 = y2_in[0]
    ar_ref[:, :] = (jnp.maximum(x2 - x1, 0.0) * jnp.maximum(y2 - y1, 0.0))

    fiota = (jax.lax.broadcasted_iota(jnp.int32, (ROWS, cols), 0) * cols
             + jax.lax.broadcasted_iota(jnp.int32, (ROWS, cols), 1))

    def body(i, _):
        s = s_ref[:, :]
        m = jnp.max(s)
        idx = jnp.min(jnp.where(s == m, fiota, n))
        oh = fiota == idx

        def selv(v):
            return jnp.max(jnp.where(oh, v, -jnp.inf))

        sx1 = selv(x1)
        sy1 = selv(y1)
        sx2 = selv(x2)
        sy2 = selv(y2)
        scl = selv(cl_in[0])
        sar = jnp.maximum(sx2 - sx1, 0.0) * jnp.maximum(sy2 - sy1, 0.0)

        ix1 = jnp.maximum(x1, sx1)
        iy1 = jnp.maximum(y1, sy1)
        ix2 = jnp.minimum(x2, sx2)
        iy2 = jnp.minimum(y2, sy2)
        inter = jnp.maximum(ix2 - ix1, 0.0) * jnp.maximum(iy2 - iy1, 0.0)
        union = jnp.maximum(ar_ref[:, :] + sar - inter, 1e-9)
        supp = (inter / union) > IOU_THRESH
        s_ref[:, :] = jnp.where(supp | oh, -1.0, s)

        valid = m > 0.0
        z = jnp.float32(0.0)
        ocls_ref[0, pl.ds(i, 1), :] = jnp.where(valid, scl, z).reshape(1, 1)
        oconf_ref[0, pl.ds(i, 1), :] = jnp.where(valid, m, z).reshape(1, 1)
        ox1_ref[0, pl.ds(i, 1), :] = jnp.where(valid, sx1, z).reshape(1, 1)
        oy1_ref[0, pl.ds(i, 1), :] = jnp.where(valid, sy1, z).reshape(1, 1)
        ox2_ref[0, pl.ds(i, 1), :] = jnp.where(valid, sx2, z).reshape(1, 1)
        oy2_ref[0, pl.ds(i, 1), :] = jnp.where(valid, sy2, z).reshape(1, 1)
        return 0

    jax.lax.fori_loop(0, TOP_K, body, 0)


def kernel(y_pred):
    b, n, c = y_pred.shape
    nchunks = n // CHUNK
    cols = n // ROWS

    dec_sds = jax.ShapeDtypeStruct((b, nchunks, 1, CHUNK), jnp.float32)
    dec_spec = pl.BlockSpec((1, 1, 1, CHUNK), lambda i, j: (i, j, 0, 0))
    dec = pl.pallas_call(
        _decode_body,
        grid=(b, nchunks),
        in_specs=[pl.BlockSpec((1, CHUNK, c), lambda i, j: (i, j, 0))],
        out_specs=[dec_spec] * 6,
        out_shape=[dec_sds] * 6,
        compiler_params=pltpu.CompilerParams(
            dimension_semantics=("parallel", "arbitrary")),
    )(y_pred)
    dec = [d.reshape(b, ROWS, cols) for d in dec]

    in_spec = pl.BlockSpec((1, ROWS, cols), lambda i: (i, 0, 0))
    out_sds = jax.ShapeDtypeStruct((b, TOP_K, 1), jnp.float32)
    out_spec = pl.BlockSpec((1, TOP_K, 1), lambda i: (i, 0, 0))
    outs = pl.pallas_call(
        _nms_body,
        grid=(b,),
        in_specs=[in_spec] * 6,
        out_specs=[out_spec] * 6,
        out_shape=[out_sds] * 6,
        scratch_shapes=[pltpu.VMEM((ROWS, cols), jnp.float32)] * 2,
        compiler_params=pltpu.CompilerParams(
            dimension_semantics=("parallel",)),
    )(*dec)
    cls, conf, x1, y1, x2, y2 = [o[..., 0] for o in outs]
    return jnp.stack([cls, conf, x1, y1, x2, y2], axis=-1)
